# baseline (speedup 1.0000x reference)
import functools

import jax
import jax.numpy as jnp
from jax import lax
from jax.experimental import pallas as pl
from jax.experimental.pallas import tpu as pltpu
from jax.experimental.pallas import tpu_sc as plsc

N = 50000
E = 800000
DEG = 16
H = 64

NC, NS = 2, 16
NW = NC * NS

DT = jnp.bfloat16
HP = 128
PW = 32

FIDX_PW = 25008
E_SRC_PAD = NW * FIDX_PW

C = 64
IPC = C * DEG // 128

BE = 2000
BN = 2000

_MESH = plsc.VectorSubcoreMesh(core_axis_name="c", subcore_axis_name="s")


@functools.partial(
    pl.kernel,
    out_type=jax.ShapeDtypeStruct((E_SRC_PAD,), jnp.int32),
    mesh=_MESH,
    compiler_params=pltpu.CompilerParams(needs_layout_passes=False),
    scratch_types=[
        pltpu.VMEM((N,), jnp.int32),
        pltpu.VMEM((FIDX_PW,), jnp.int32),
        pltpu.VMEM((FIDX_PW,), jnp.int32),
    ],
)
def _sc_fidx(fnode_hbm, src_hbm, out_hbm, fnode_v, src_v, out_v):
    wid = lax.axis_index("s") * NC + lax.axis_index("c")
    base = wid * FIDX_PW
    pltpu.sync_copy(fnode_hbm, fnode_v)
    pltpu.sync_copy(src_hbm.at[pl.ds(base, FIDX_PW)], src_v)

    def grp(g, carry):
        s16 = src_v[pl.ds(g * 16, 16)]
        out_v[pl.ds(g * 16, 16)] = plsc.load_gather(fnode_v, [s16])
        return carry

    lax.fori_loop(0, FIDX_PW // 16, grp, 0)
    pltpu.sync_copy(out_v, out_hbm.at[pl.ds(base, FIDX_PW)])


def _make_gather_sum(r_pad, k_per_w):
    assert r_pad == NW * C * k_per_w and k_per_w % 2 == 0

    @functools.partial(
        pl.kernel,
        out_type=jax.ShapeDtypeStruct((r_pad, PW), jnp.int32),
        mesh=_MESH,
        compiler_params=pltpu.CompilerParams(
            needs_layout_passes=False, use_tc_tiling_on_sc=False),
        scratch_types=[
            pltpu.VMEM((2, IPC, 128), jnp.int32),
            pltpu.VMEM((2, C * DEG, PW), jnp.int32),
            pltpu.VMEM((2, C, PW), jnp.int32),
            pltpu.SemaphoreType.DMA,
            pltpu.SemaphoreType.DMA,
            pltpu.SemaphoreType.DMA,
            pltpu.SemaphoreType.DMA,
            pltpu.SemaphoreType.DMA,
            pltpu.SemaphoreType.DMA,
        ],
    )
    def gather_sum(src_hbm, idx_hbm, out_hbm, o_idx, o_rows, o_out,
                   sidx0, sidx1, srow0, srow1, sout0, sout1):
        wid = lax.axis_index("s") * NC + lax.axis_index("c")
        c0 = wid * k_per_w
        sidx = (sidx0, sidx1)
        srow = (srow0, srow1)
        sout = (sout0, sout1)

        def idx_cp(c, b):
            return pltpu.make_async_copy(
                idx_hbm.at[pl.ds(c * IPC, IPC)], o_idx.at[b], sidx[b])

        def row_cp(b, g):
            return pltpu.make_async_copy(
                src_hbm.at[o_idx.at[b, g]],
                o_rows.at[b, pl.ds(g * 128, 128)], srow[b])

        def out_cp(c, b):
            return pltpu.make_async_copy(
                o_out.at[b], out_hbm.at[pl.ds(c * C, C)], sout[b])

        def fire_rows(b):
            for g in range(IPC):
                row_cp(b, g).start()

        def wait_rows(b):
            for g in range(IPC):
                row_cp(b, g).wait()

        def compute(b):
            rows = o_rows.at[b]
            out = o_out.at[b]
            f32 = jnp.float32

            def unpk(r):
                w0 = rows[r, pl.ds(0, 16)]
                w1 = rows[r, pl.ds(16, 16)]
                return (plsc.bitcast(jnp.left_shift(w0, 16), f32),
                        plsc.bitcast(jnp.left_shift(w1, 16), f32),
                        plsc.bitcast(w0, f32),
                        plsc.bitcast(w1, f32))

            fmt = plsc.PackFormat.INTERLEAVED

            def edge(i, carry):
                base = i * DEG
                acc = list(unpk(base))
                for kk in range(1, DEG):
                    vals = unpk(base + kk)
                    for d in range(4):
                        acc[d] = acc[d] + vals[d]
                out[i, pl.ds(0, 16)] = plsc.bitcast(
                    plsc.pack(acc[0], acc[2], format=fmt), jnp.int32)
                out[i, pl.ds(16, 16)] = plsc.bitcast(
                    plsc.pack(acc[1], acc[3], format=fmt), jnp.int32)
                return carry

            lax.fori_loop(0, C, edge, 0)

        kh = k_per_w // 2
        idx_cp(c0, 0).start()
        idx_cp(c0 + 1, 1).start()
        idx_cp(c0, 0).wait()
        fire_rows(0)

        def step(j, carry):
            ca = c0 + 2 * j
            cb = ca + 1

            idx_cp(cb, 1).wait()
            fire_rows(1)

            wait_rows(0)

            @pl.when(j < kh - 1)
            def _():
                idx_cp(ca + 2, 0).start()

            @pl.when(j > 0)
            def _():
                out_cp(ca - 2, 0).wait()

            compute(0)
            out_cp(ca, 0).start()

            @pl.when(j < kh - 1)
            def _():
                idx_cp(ca + 2, 0).wait()
                fire_rows(0)

            wait_rows(1)

            @pl.when(j < kh - 1)
            def _():
                idx_cp(cb + 2, 1).start()

            @pl.when(j > 0)
            def _():
                out_cp(cb - 2, 1).wait()

            compute(1)
            out_cp(cb, 1).start()
            return carry

        lax.fori_loop(0, kh, step, 0)
        out_cp(c0 + k_per_w - 2, 0).wait()
        out_cp(c0 + k_per_w - 1, 1).wait()

    return gather_sum


K_E = (E + NW * C - 1) // (NW * C)
K_E += K_E % 2
E_PAD = NW * C * K_E
K_N = (N + NW * C - 1) // (NW * C)
K_N += K_N % 2
N_PAD = NW * C * K_N

_gs_edges = _make_gather_sum(E_PAD, K_E)
_gs_nodes = _make_gather_sum(N_PAD, K_N)


_DN_T = (((0,), (0,)), ((), ()))
_DN = (((1,), (0,)), ((), ()))


def _pack_h(h, b):
    bits = lax.bitcast_convert_type(h.astype(jnp.bfloat16), jnp.uint16)
    lo = bits[:, 0:32].astype(jnp.int32)
    hi = bits[:, 32:64].astype(jnp.int32)
    return jnp.bitwise_or(lo, jnp.left_shift(hi, 16))


def _unpack_h(w):
    f32 = jnp.float32
    lo = lax.bitcast_convert_type(jnp.left_shift(w, 16), f32)
    hi = lax.bitcast_convert_type(
        jnp.bitwise_and(w, jnp.int32(-65536)), f32)
    return jnp.concatenate([lo, hi], axis=1)


def _onehot_t(idx_row, v, b):
    ii = lax.broadcasted_iota(jnp.int32, (v, b), 0)
    return (jnp.broadcast_to(idx_row, (v, b)) == ii).astype(DT)


def _fold_body(ea_ref, eb_ref, w0_ref, wh_ref, wo1_ref,
               s1_ref, s2_ref, t1_ref, t2_ref, t3_ref):
    ea = ea_ref[...]
    eb = eb_ref[...]
    f32 = jnp.float32
    s1_ref[...] = lax.dot_general(ea, w0_ref[0:64, :], _DN, preferred_element_type=f32).astype(DT)
    s2_ref[...] = lax.dot_general(eb, w0_ref[64:80, :], _DN, preferred_element_type=f32).astype(DT)
    t1_ref[...] = lax.dot_general(ea, wh_ref[0:64, :], _DN, preferred_element_type=f32).astype(DT)
    t2_ref[...] = lax.dot_general(eb, wh_ref[64:80, :], _DN, preferred_element_type=f32).astype(DT)
    t3_ref[...] = lax.dot_general(ea, wo1_ref[0:64, :], _DN, preferred_element_type=f32).astype(DT)


def _embed_body(fidx_ref, bond_ref, s1_ref, s2_ref, t1_ref, t2_ref,
                b0_ref, bh_ref, h0_ref, u_ref):
    f32 = jnp.float32
    oa = _onehot_t(fidx_ref[0], 128, BE)
    ob = _onehot_t(bond_ref[0], 8, BE)
    pre_h = (lax.dot_general(oa, s1_ref[...], _DN_T, preferred_element_type=f32)
             + lax.dot_general(ob, s2_ref[...], _DN_T, preferred_element_type=f32)
             + b0_ref[...])
    pre_u = (lax.dot_general(oa, t1_ref[...], _DN_T, preferred_element_type=f32)
             + lax.dot_general(ob, t2_ref[...], _DN_T, preferred_element_type=f32)
             + bh_ref[...])
    h0 = jnp.maximum(pre_h, 0.0)
    rid = lax.broadcasted_iota(jnp.int32, (BE, 1), 0)
    is0 = jnp.logical_and(pl.program_id(0) == 0, rid == 0)
    h0_ref[...] = _pack_h(jnp.where(is0, 0.0, h0), BE)
    u_ref[...] = pre_u.astype(DT)


def _round_body(u_ref, nei_ref, whh_ref, h_ref):
    f32 = jnp.float32
    acc = lax.dot_general(_unpack_h(nei_ref[...]), whh_ref[...], _DN,
                          preferred_element_type=f32)
    h = jnp.maximum(acc + u_ref[...].astype(f32), 0.0)
    rid = lax.broadcasted_iota(jnp.int32, (BE, 1), 0)
    is0 = jnp.logical_and(pl.program_id(0) == 0, rid == 0)
    h_ref[...] = _pack_h(jnp.where(is0, 0.0, h), BE)


def _out_body(fnode_ref, nnode_ref, t3_ref, w1h_ref, w2_ref,
              bo1_ref, bo2_ref, o_ref):
    f32 = jnp.float32
    oa = _onehot_t(fnode_ref[0], 128, BN)
    z = (lax.dot_general(oa, t3_ref[...], _DN_T, preferred_element_type=f32)
         + lax.dot_general(_unpack_h(nnode_ref[...]), w1h_ref[...], _DN, preferred_element_type=f32)
         + bo1_ref[...])
    z = jnp.maximum(z, 0.0)
    o_ref[...] = (lax.dot_general(z, w2_ref[...], _DN, preferred_element_type=f32)
                  + bo2_ref[...])


def _full(shape):
    return pl.BlockSpec(shape, lambda i: tuple(0 for _ in shape))


def kernel(fnode, fmess_src, fmess_bond, agraph, bgraph,
           E_a, E_b, W0, b0, Wh, bh, Wo1, bo1, Wo2, bo2):
    i32 = jnp.int32
    f32 = jnp.float32

    ea_pad = jnp.concatenate([E_a, jnp.zeros((28, 64), f32)], axis=0)
    eb_pad = jnp.concatenate([E_b, jnp.zeros((2, 16), f32)], axis=0)
    src_pad = jnp.concatenate(
        [fmess_src.astype(i32), jnp.zeros((E_SRC_PAD - E,), i32)])
    bg_pad = jnp.concatenate(
        [bgraph.astype(i32), jnp.zeros((E_PAD - E, DEG), i32)]
    ).reshape(E_PAD * DEG // 128, 128)
    ag_pad = jnp.concatenate(
        [agraph.astype(i32), jnp.zeros((N_PAD - N, DEG), i32)]
    ).reshape(N_PAD * DEG // 128, 128)
    bond3 = fmess_bond.astype(i32).reshape(E // BE, 1, BE)
    fnode3 = fnode.astype(i32).reshape(N // BN, 1, BN)
    whh = Wh[80:144, :]
    w1h = Wo1[64:128, :]
    b0r = b0.reshape(1, H)
    bhr = bh.reshape(1, H)
    bo1r = bo1.reshape(1, H)
    bo2r = bo2.reshape(1, H)

    s1, s2, t1, t2, t3 = pl.pallas_call(
        _fold_body,
        out_shape=[
            jax.ShapeDtypeStruct((128, H), DT),
            jax.ShapeDtypeStruct((8, H), DT),
            jax.ShapeDtypeStruct((128, H), DT),
            jax.ShapeDtypeStruct((8, H), DT),
            jax.ShapeDtypeStruct((128, H), DT),
        ],
    )(ea_pad, eb_pad, W0, Wh, Wo1)

    fidx = _sc_fidx(fnode.astype(i32), src_pad)
    fidx3 = fidx[:E].reshape(E // BE, 1, BE)

    nb_e = E // BE
    h0, u = pl.pallas_call(
        _embed_body,
        grid=(nb_e,),
        in_specs=[
            pl.BlockSpec((1, 1, BE), lambda i: (i, 0, 0)),
            pl.BlockSpec((1, 1, BE), lambda i: (i, 0, 0)),
            _full((128, H)), _full((8, H)), _full((128, H)), _full((8, H)),
            _full((1, H)), _full((1, H)),
        ],
        out_specs=[pl.BlockSpec((BE, PW), lambda i: (i, 0)),
                   pl.BlockSpec((BE, H), lambda i: (i, 0))],
        out_shape=[jax.ShapeDtypeStruct((E, PW), jnp.int32),
                   jax.ShapeDtypeStruct((E, H), DT)],
    )(fidx3, bond3, s1, s2, t1, t2, b0r, bhr)

    round_call = pl.pallas_call(
        _round_body,
        grid=(nb_e,),
        in_specs=[
            pl.BlockSpec((BE, H), lambda i: (i, 0)),
            pl.BlockSpec((BE, PW), lambda i: (i, 0)),
            _full((H, H)),
        ],
        out_specs=pl.BlockSpec((BE, PW), lambda i: (i, 0)),
        out_shape=jax.ShapeDtypeStruct((E, PW), jnp.int32),
    )

    h = h0
    for _ in range(2):
        nei = _gs_edges(h, bg_pad)
        h = round_call(u, nei, whh)

    nnode = _gs_nodes(h, ag_pad)
    hatom = pl.pallas_call(
        _out_body,
        grid=(N // BN,),
        in_specs=[
            pl.BlockSpec((1, 1, BN), lambda i: (i, 0, 0)),
            pl.BlockSpec((BN, PW), lambda i: (i, 0)),
            _full((128, H)), _full((H, H)), _full((H, H)),
            _full((1, H)), _full((1, H)),
        ],
        out_specs=pl.BlockSpec((BN, H), lambda i: (i, 0)),
        out_shape=jax.ShapeDtypeStruct((N, H), f32),
    )(fnode3, nnode, t3, w1h, Wo2, bo1r, bo2r)

    return hatom


# speedup vs baseline: 1.1346x; 1.1346x over previous
import functools

import jax
import jax.numpy as jnp
from jax import lax
from jax.experimental import pallas as pl
from jax.experimental.pallas import tpu as pltpu
from jax.experimental.pallas import tpu_sc as plsc

N = 50000
E = 800000
DEG = 16
H = 64

NC, NS = 2, 16
NW = NC * NS

DT = jnp.bfloat16
HP = 128
PW = 32

FIDX_PW = 25008
E_SRC_PAD = NW * FIDX_PW

C = 64
IPC = C * DEG // 128

BE = 2000
BN = 2000

_MESH = plsc.VectorSubcoreMesh(core_axis_name="c", subcore_axis_name="s")


@functools.partial(
    pl.kernel,
    out_type=jax.ShapeDtypeStruct((E_SRC_PAD,), jnp.int32),
    mesh=_MESH,
    compiler_params=pltpu.CompilerParams(needs_layout_passes=False),
    scratch_types=[
        pltpu.VMEM((N,), jnp.int32),
        pltpu.VMEM((FIDX_PW,), jnp.int32),
        pltpu.VMEM((FIDX_PW,), jnp.int32),
    ],
)
def _sc_fidx(fnode_hbm, src_hbm, out_hbm, fnode_v, src_v, out_v):
    wid = lax.axis_index("s") * NC + lax.axis_index("c")
    base = wid * FIDX_PW
    pltpu.sync_copy(fnode_hbm, fnode_v)
    pltpu.sync_copy(src_hbm.at[pl.ds(base, FIDX_PW)], src_v)

    def grp(g, carry):
        s16 = src_v[pl.ds(g * 16, 16)]
        out_v[pl.ds(g * 16, 16)] = plsc.load_gather(fnode_v, [s16])
        return carry

    lax.fori_loop(0, FIDX_PW // 16, grp, 0)
    pltpu.sync_copy(out_v, out_hbm.at[pl.ds(base, FIDX_PW)])


def _make_gather_sum(r_pad, k_per_w):
    assert r_pad == NW * C * k_per_w and k_per_w % 2 == 0

    @functools.partial(
        pl.kernel,
        out_type=jax.ShapeDtypeStruct((r_pad, HP), jnp.float32),
        mesh=_MESH,
        compiler_params=pltpu.CompilerParams(
            needs_layout_passes=False, use_tc_tiling_on_sc=False),
        scratch_types=[
            pltpu.VMEM((2, IPC, 128), jnp.int32),
            pltpu.VMEM((2, C * DEG, PW), jnp.int32),
            pltpu.VMEM((2, C, HP), jnp.float32),
            pltpu.SemaphoreType.DMA,
            pltpu.SemaphoreType.DMA,
            pltpu.SemaphoreType.DMA,
            pltpu.SemaphoreType.DMA,
            pltpu.SemaphoreType.DMA,
            pltpu.SemaphoreType.DMA,
        ],
    )
    def gather_sum(src_hbm, idx_hbm, out_hbm, o_idx, o_rows, o_out,
                   sidx0, sidx1, srow0, srow1, sout0, sout1):
        wid = lax.axis_index("s") * NC + lax.axis_index("c")
        c0 = wid * k_per_w
        sidx = (sidx0, sidx1)
        srow = (srow0, srow1)
        sout = (sout0, sout1)

        def idx_cp(c, b):
            return pltpu.make_async_copy(
                idx_hbm.at[pl.ds(c * IPC, IPC)], o_idx.at[b], sidx[b])

        def row_cp(b, g):
            return pltpu.make_async_copy(
                src_hbm.at[o_idx.at[b, g]],
                o_rows.at[b, pl.ds(g * 128, 128)], srow[b])

        def out_cp(c, b):
            return pltpu.make_async_copy(
                o_out.at[b], out_hbm.at[pl.ds(c * C, C)], sout[b])

        def fire_rows(b):
            for g in range(IPC):
                row_cp(b, g).start()

        def wait_rows(b):
            for g in range(IPC):
                row_cp(b, g).wait()

        def compute(b):
            rows = o_rows.at[b]
            out = o_out.at[b]
            f32 = jnp.float32

            def unpk(r):
                w0 = rows[r, pl.ds(0, 16)]
                w1 = rows[r, pl.ds(16, 16)]
                return (plsc.bitcast(jnp.left_shift(w0, 16), f32),
                        plsc.bitcast(jnp.left_shift(w1, 16), f32),
                        plsc.bitcast(w0, f32),
                        plsc.bitcast(w1, f32))

            def edge(i, carry):
                base = i * DEG
                acc = list(unpk(base))
                for kk in range(1, DEG):
                    vals = unpk(base + kk)
                    for d in range(4):
                        acc[d] = acc[d] + vals[d]
                for d in range(4):
                    out[i, pl.ds(16 * d, 16)] = acc[d]
                return carry

            lax.fori_loop(0, C, edge, 0)

        zz = jnp.zeros((16,), jnp.float32)

        def zinit(i, carry):
            for b in range(2):
                for d in range(4, 8):
                    o_out[b, i, pl.ds(16 * d, 16)] = zz
            return carry

        lax.fori_loop(0, C, zinit, 0)

        kh = k_per_w // 2
        idx_cp(c0, 0).start()
        idx_cp(c0 + 1, 1).start()
        idx_cp(c0, 0).wait()
        fire_rows(0)

        def step(j, carry):
            ca = c0 + 2 * j
            cb = ca + 1

            idx_cp(cb, 1).wait()
            fire_rows(1)

            wait_rows(0)

            @pl.when(j < kh - 1)
            def _():
                idx_cp(ca + 2, 0).start()

            @pl.when(j > 0)
            def _():
                out_cp(ca - 2, 0).wait()

            compute(0)
            out_cp(ca, 0).start()

            @pl.when(j < kh - 1)
            def _():
                idx_cp(ca + 2, 0).wait()
                fire_rows(0)

            wait_rows(1)

            @pl.when(j < kh - 1)
            def _():
                idx_cp(cb + 2, 1).start()

            @pl.when(j > 0)
            def _():
                out_cp(cb - 2, 1).wait()

            compute(1)
            out_cp(cb, 1).start()
            return carry

        lax.fori_loop(0, kh, step, 0)
        out_cp(c0 + k_per_w - 2, 0).wait()
        out_cp(c0 + k_per_w - 1, 1).wait()

    return gather_sum


K_E = (E + NW * C - 1) // (NW * C)
K_E += K_E % 2
E_PAD = NW * C * K_E
K_N = (N + NW * C - 1) // (NW * C)
K_N += K_N % 2
N_PAD = NW * C * K_N

_gs_edges = _make_gather_sum(E_PAD, K_E)
_gs_nodes = _make_gather_sum(N_PAD, K_N)


_DN_T = (((0,), (0,)), ((), ()))
_DN = (((1,), (0,)), ((), ()))


def _pack_h(h, b):
    bits = lax.bitcast_convert_type(h.astype(jnp.bfloat16), jnp.uint16)
    lo = bits[:, 0:32].astype(jnp.int32)
    hi = bits[:, 32:64].astype(jnp.int32)
    return jnp.bitwise_or(lo, jnp.left_shift(hi, 16))


def _onehot_t(idx_row, v, b):
    ii = lax.broadcasted_iota(jnp.int32, (v, b), 0)
    return (jnp.broadcast_to(idx_row, (v, b)) == ii).astype(DT)


def _embed_body(fidx_ref, bond_ref, ea_ref, eb_ref, w0_ref, wh_ref,
                b0_ref, bh_ref, h0_ref, u_ref):
    f32 = jnp.float32
    ea = ea_ref[...]
    eb = eb_ref[...]
    s1 = lax.dot_general(ea, w0_ref[0:64, :], _DN, preferred_element_type=f32).astype(DT)
    s2 = lax.dot_general(eb, w0_ref[64:80, :], _DN, preferred_element_type=f32).astype(DT)
    t1 = lax.dot_general(ea, wh_ref[0:64, :], _DN, preferred_element_type=f32).astype(DT)
    t2 = lax.dot_general(eb, wh_ref[64:80, :], _DN, preferred_element_type=f32).astype(DT)
    oa = _onehot_t(fidx_ref[0], 128, BE)
    ob = _onehot_t(bond_ref[0], 8, BE)
    pre_h = (lax.dot_general(oa, s1, _DN_T, preferred_element_type=f32)
             + lax.dot_general(ob, s2, _DN_T, preferred_element_type=f32)
             + b0_ref[...])
    pre_u = (lax.dot_general(oa, t1, _DN_T, preferred_element_type=f32)
             + lax.dot_general(ob, t2, _DN_T, preferred_element_type=f32)
             + bh_ref[...])
    h0 = jnp.maximum(pre_h, 0.0)
    rid = lax.broadcasted_iota(jnp.int32, (BE, 1), 0)
    is0 = jnp.logical_and(pl.program_id(0) == 0, rid == 0)
    h0_ref[...] = _pack_h(jnp.where(is0, 0.0, h0), BE)
    u_ref[...] = pre_u.astype(DT)


def _round_body(u_ref, nei_ref, whh_ref, h_ref):
    f32 = jnp.float32
    acc = lax.dot_general(nei_ref[:, 0:64], whh_ref[...], _DN,
                          preferred_element_type=f32)
    h = jnp.maximum(acc + u_ref[...].astype(f32), 0.0)
    rid = lax.broadcasted_iota(jnp.int32, (BE, 1), 0)
    is0 = jnp.logical_and(pl.program_id(0) == 0, rid == 0)
    h_ref[...] = _pack_h(jnp.where(is0, 0.0, h), BE)


def _out_body(fnode_ref, nnode_ref, ea_ref, wo1_ref, w2_ref,
              bo1_ref, bo2_ref, o_ref):
    f32 = jnp.float32
    t3 = lax.dot_general(ea_ref[...], wo1_ref[0:64, :], _DN,
                         preferred_element_type=f32).astype(DT)
    oa = _onehot_t(fnode_ref[0], 128, BN)
    z = (lax.dot_general(oa, t3, _DN_T, preferred_element_type=f32)
         + lax.dot_general(nnode_ref[:, 0:64], wo1_ref[64:128, :], _DN,
                           preferred_element_type=f32)
         + bo1_ref[...])
    z = jnp.maximum(z, 0.0)
    o_ref[...] = (lax.dot_general(z, w2_ref[...], _DN, preferred_element_type=f32)
                  + bo2_ref[...])


def _full(shape):
    return pl.BlockSpec(shape, lambda i: tuple(0 for _ in shape))


def kernel(fnode, fmess_src, fmess_bond, agraph, bgraph,
           E_a, E_b, W0, b0, Wh, bh, Wo1, bo1, Wo2, bo2):
    i32 = jnp.int32
    f32 = jnp.float32

    ea_pad = jnp.concatenate([E_a, jnp.zeros((28, 64), f32)], axis=0)
    eb_pad = jnp.concatenate([E_b, jnp.zeros((2, 16), f32)], axis=0)
    src_pad = jnp.concatenate(
        [fmess_src.astype(i32), jnp.zeros((E_SRC_PAD - E,), i32)])
    bg_pad = jnp.concatenate([
        bgraph.astype(i32).reshape(E * DEG // 128, 128),
        jnp.zeros(((E_PAD - E) * DEG // 128, 128), i32)])
    ag_pad = jnp.concatenate([
        agraph.astype(i32).reshape(N * DEG // 128, 128),
        jnp.zeros(((N_PAD - N) * DEG // 128, 128), i32)])
    bond3 = fmess_bond.astype(i32).reshape(E // BE, 1, BE)
    fnode3 = fnode.astype(i32).reshape(N // BN, 1, BN)
    whh = Wh[80:144, :]
    b0r = b0.reshape(1, H)
    bhr = bh.reshape(1, H)
    bo1r = bo1.reshape(1, H)
    bo2r = bo2.reshape(1, H)

    fidx = _sc_fidx(fnode.astype(i32), src_pad)
    fidx3 = fidx[:E].reshape(E // BE, 1, BE)

    nb_e = E // BE
    h0, u = pl.pallas_call(
        _embed_body,
        grid=(nb_e,),
        in_specs=[
            pl.BlockSpec((1, 1, BE), lambda i: (i, 0, 0)),
            pl.BlockSpec((1, 1, BE), lambda i: (i, 0, 0)),
            _full((128, H)), _full((8, 16)), _full((80, H)), _full((144, H)),
            _full((1, H)), _full((1, H)),
        ],
        out_specs=[pl.BlockSpec((BE, PW), lambda i: (i, 0)),
                   pl.BlockSpec((BE, H), lambda i: (i, 0))],
        out_shape=[jax.ShapeDtypeStruct((E, PW), i32),
                   jax.ShapeDtypeStruct((E, H), DT)],
    )(fidx3, bond3, ea_pad, eb_pad, W0, Wh, b0r, bhr)

    round_call = pl.pallas_call(
        _round_body,
        grid=(nb_e,),
        in_specs=[
            pl.BlockSpec((BE, H), lambda i: (i, 0)),
            pl.BlockSpec((BE, HP), lambda i: (i, 0)),
            _full((H, H)),
        ],
        out_specs=pl.BlockSpec((BE, PW), lambda i: (i, 0)),
        out_shape=jax.ShapeDtypeStruct((E, PW), i32),
    )

    h = h0
    for _ in range(2):
        nei = _gs_edges(h, bg_pad)
        h = round_call(u, nei, whh)

    nnode = _gs_nodes(h, ag_pad)
    hatom = pl.pallas_call(
        _out_body,
        grid=(N // BN,),
        in_specs=[
            pl.BlockSpec((1, 1, BN), lambda i: (i, 0, 0)),
            pl.BlockSpec((BN, HP), lambda i: (i, 0)),
            _full((128, H)), _full((128, H)), _full((H, H)),
            _full((1, H)), _full((1, H)),
        ],
        out_specs=pl.BlockSpec((BN, H), lambda i: (i, 0)),
        out_shape=jax.ShapeDtypeStruct((N, H), f32),
    )(fnode3, nnode, ea_pad, Wo1, Wo2, bo1r, bo2r)

    return hatom
